# baseline (speedup 1.0000x reference)
import functools

import jax
import jax.numpy as jnp
from jax.experimental import pallas as pl
from jax.experimental.pallas import tpu as pltpu
from jax.experimental.pallas import tpu_sc as plsc

_B, _S, _H = 4, 4096, 1024
_F, _T = 4, 8
_OUT = 1024
_NC = 16
_NTILES = 32
_RPT = _B * _S // _NTILES
_TPB = _NTILES // _B
_PIECE = 32
_NPIECE = _RPT // _PIECE
_HW = _H // 16
_TOKW = _F * _T * 2 * 16

_mesh = plsc.VectorSubcoreMesh(core_axis_name="c", subcore_axis_name="s")


@functools.partial(
    pl.kernel,
    mesh=_mesh,
    out_type=[
        jax.ShapeDtypeStruct((_NTILES * _NC * _H,), jnp.float32),
        jax.ShapeDtypeStruct((_NTILES * _NC * 16,), jnp.float32),
    ],
    scratch_types=[
        pltpu.VMEM((2 * _PIECE * _H,), jnp.float32),
        pltpu.VMEM((_NC * _H,), jnp.float32),
        pltpu.VMEM((_RPT,), jnp.int32),
        pltpu.VMEM((_TOKW,), jnp.int32),
        pltpu.VMEM((_NC * 16,), jnp.float32),
        pltpu.SemaphoreType.DMA,
    ],
)
def _sc_pool(hid_ref, tok_ref, part_ref, cnt_ref,
             buf, acc, combo_ref, tokv, cntv, sem):
    cid = jax.lax.axis_index("c")
    sid = jax.lax.axis_index("s")
    wid = sid * 2 + cid
    row0 = wid * _RPT
    s0 = (wid % _TPB) * _RPT
    b = wid // _TPB

    pltpu.async_copy(hid_ref.at[pl.ds(row0 * _H, _PIECE * _H)],
                     buf.at[pl.ds(0, _PIECE * _H)], sem)

    pltpu.sync_copy(tok_ref.at[pl.ds(b * _TOKW, _TOKW)], tokv)

    zero16 = jnp.zeros((16,), jnp.float32)

    def _zbody(i, carry):
        for j in range(16):
            acc[pl.ds(i * 256 + j * 16, 16)] = zero16
        return carry

    jax.lax.fori_loop(0, _NC * _H // 256, _zbody, 0)

    lane = jax.lax.iota(jnp.int32, 16)

    def _cbody(v, cnts):
        pos = lane + jnp.broadcast_to(s0 + v * 16, (16,))
        combo = jnp.zeros((16,), jnp.int32)
        for f in range(_F):
            cov = None
            for t in range(_T):
                base = ((f * _T + t) * 2) * 16
                st = tokv[pl.ds(base, 16)]
                en = tokv[pl.ds(base + 16, 16)]
                c1 = (pos >= st) & (pos < en)
                cov = c1 if cov is None else (cov | c1)
            combo = combo + jnp.where(cov, 1 << f, 0)
        combo_ref[pl.ds(v * 16, 16)] = combo << 10
        return tuple(
            cnts[c] + jnp.where(combo == c, 1.0, 0.0) for c in range(_NC))

    cnts = jax.lax.fori_loop(0, _RPT // 16, _cbody, (zero16,) * _NC)
    for c in range(_NC):
        cntv[pl.ds(c * 16, 16)] = cnts[c]

    def _pbody(p, carry):
        nslot = (p + 1) % 2

        @pl.when(p + 1 < _NPIECE)
        def _():
            pltpu.async_copy(
                hid_ref.at[pl.ds((row0 + (p + 1) * _PIECE) * _H, _PIECE * _H)],
                buf.at[pl.ds(nslot * _PIECE * _H, _PIECE * _H)], sem)

        pltpu.make_async_copy(hid_ref.at[pl.ds(row0 * _H, _PIECE * _H)],
                              buf.at[pl.ds(0, _PIECE * _H)], sem).wait()
        boff = (p % 2) * _PIECE * _H

        def _gbody(l, c2):
            cvec = combo_ref[pl.ds((p * 2 + l) * 16, 16)]
            gb = boff + l * 16 * _H
            for i in range(16):
                off = cvec[i]
                rb = gb + i * _H
                for j in range(_HW):
                    plsc.addupdate(acc.at[pl.ds(off + j * 16, 16)],
                                   buf[pl.ds(rb + j * 16, 16)])
            return c2

        jax.lax.fori_loop(0, 2, _gbody, 0)
        return carry

    jax.lax.fori_loop(0, _NPIECE, _pbody, 0)

    pltpu.sync_copy(acc, part_ref.at[pl.ds(wid * _NC * _H, _NC * _H)])
    pltpu.sync_copy(cntv, cnt_ref.at[pl.ds(wid * _NC * 16, _NC * 16)])


def _proj_body(ps_ref, cv_ref, w_ref, b_ref, out_ref):
    lane = jax.lax.broadcasted_iota(jnp.int32, (1, _TPB * _NC), 1) % _NC
    rows = []
    for f in range(8):
        if f < _F:
            rows.append(((lane >> f) & 1).astype(jnp.float32))
        else:
            rows.append(jnp.zeros((1, _TPB * _NC), jnp.float32))
    sel = jnp.concatenate(rows, axis=0)
    for b in range(_B):
        psum = jnp.dot(sel, ps_ref[b], preferred_element_type=jnp.float32)
        dsum = jnp.dot(sel, cv_ref[b], preferred_element_type=jnp.float32)
        denom = jnp.maximum(jnp.sum(dsum, axis=1, keepdims=True), 1.0)
        pooled = psum / denom
        y = jnp.dot(pooled, w_ref[...], preferred_element_type=jnp.float32)
        y = jnp.tanh(y + b_ref[...])
        out_ref[b] = y[:_F]


_proj = pl.pallas_call(
    _proj_body,
    out_shape=jax.ShapeDtypeStruct((_B, _F, _OUT), jnp.float32),
)


def kernel(hidden, token_idxs, W, b):
    tok = token_idxs.astype(jnp.int32)
    tokb = jnp.broadcast_to(
        jnp.transpose(tok, (1, 0, 2, 3))[..., None],
        (_B, _F, _T, 2, 16)).reshape(-1)
    hid = hidden.reshape(-1)
    part, cnt = _sc_pool(hid, tokb)
    ps = part.reshape(_B, _TPB * _NC, _H)
    cv = cnt.reshape(_B, _TPB * _NC, 16)
    return _proj(ps, cv, W, b.reshape(1, _OUT))


# speedup vs baseline: 8.4132x; 8.4132x over previous
import jax
import jax.numpy as jnp
from jax.experimental import pallas as pl
from jax.experimental.pallas import tpu as pltpu

_B, _S, _H = 4, 4096, 1024
_F, _T = 4, 8
_OUT = 1024
_FP = 8
_SB = 2048
_NS = _S // _SB


def _body(tok_ref, hid_ref, w_ref, b_ref, out_ref, acc_ref, cacc_ref):
    bi = pl.program_id(0)
    si = pl.program_id(1)

    @pl.when(si == 0)
    def _():
        acc_ref[...] = jnp.zeros_like(acc_ref)
        cacc_ref[...] = jnp.zeros_like(cacc_ref)

    pos = si * _SB + jax.lax.broadcasted_iota(jnp.int32, (1, _SB), 1)
    rows = []
    for f in range(_FP):
        m = jnp.zeros((1, _SB), jnp.bool_)
        if f < _F:
            for t in range(_T):
                st = tok_ref[f, bi, t, 0]
                en = tok_ref[f, bi, t, 1]
                m = m | ((pos >= st) & (pos < en))
        rows.append(m.astype(jnp.float32))
    mask = jnp.concatenate(rows, axis=0)

    h = hid_ref[0]
    acc_ref[...] += jnp.dot(mask, h, preferred_element_type=jnp.float32)
    cacc_ref[...] += jnp.sum(mask, axis=1, keepdims=True)

    @pl.when(si == _NS - 1)
    def _():
        denom = jnp.maximum(cacc_ref[..., 0:1], 1.0)
        pooled = acc_ref[...] / denom
        y = jnp.dot(pooled, w_ref[...], preferred_element_type=jnp.float32)
        y = jnp.tanh(y + b_ref[...])
        out_ref[0] = y[:_F]


_fused = pl.pallas_call(
    _body,
    grid=(_B, _NS),
    in_specs=[
        pl.BlockSpec(memory_space=pltpu.SMEM),
        pl.BlockSpec((1, _SB, _H), lambda b, s: (b, s, 0)),
        pl.BlockSpec((_H, _OUT), lambda b, s: (0, 0)),
        pl.BlockSpec((1, _OUT), lambda b, s: (0, 0)),
    ],
    out_specs=pl.BlockSpec((1, _F, _OUT), lambda b, s: (b, 0, 0)),
    out_shape=jax.ShapeDtypeStruct((_B, _F, _OUT), jnp.float32),
    scratch_shapes=[
        pltpu.VMEM((_FP, _H), jnp.float32),
        pltpu.VMEM((_FP, 128), jnp.float32),
    ],
    compiler_params=pltpu.CompilerParams(
        dimension_semantics=("parallel", "arbitrary"),
    ),
)


def kernel(hidden, token_idxs, W, b):
    tok = token_idxs.astype(jnp.int32)
    return _fused(tok, hidden, W, b.reshape(1, _OUT))
